# Initial kernel scaffold; baseline (speedup 1.0000x reference)
import jax
import jax.numpy as jnp
from jax.experimental import pallas as pl
from jax.experimental.pallas import tpu as pltpu

_BATCH_BLK = 256


def _body(x_ref, tok_ref, pos_ref, out_ref):
    xb = x_ref[...]
    Bb, S = xb.shape
    V, D = tok_ref.shape
    oh = (xb.reshape(Bb * S, 1)
          == jax.lax.broadcasted_iota(jnp.int32, (1, V), 1)).astype(jnp.float32)
    tok = tok_ref[...]
    t = jax.lax.dot_general(oh, tok, (((1,), (0,)), ((), ())),
                            preferred_element_type=jnp.float32)
    out_ref[...] = t.reshape(Bb, S, D) + pos_ref[...][None]


def kernel(x, token_emb, pos_emb):
    x = x.astype(jnp.int32)
    B, S = x.shape
    V, D = token_emb.shape
    grid = (B // _BATCH_BLK,)
    return pl.pallas_call(
        _body,
        grid=grid,
        in_specs=[
            pl.BlockSpec((_BATCH_BLK, S), lambda i: (i, 0)),
            pl.BlockSpec((V, D), lambda i: (0, 0)),
            pl.BlockSpec((S, D), lambda i: (0, 0)),
        ],
        out_specs=pl.BlockSpec((_BATCH_BLK, S, D), lambda i: (i, 0, 0)),
        out_shape=jax.ShapeDtypeStruct((B, S, D), jnp.float32),
    )(x, token_emb, pos_emb)


# speedup vs baseline: 20.7017x; 20.7017x over previous
import jax
import jax.numpy as jnp
from jax.experimental import pallas as pl
from jax.experimental.pallas import tpu as pltpu

_BATCH_BLK = 256


def _body(x_ref, tok_ref, pos_ref, out_ref):
    xb = x_ref[...]
    Bb, S = xb.shape
    V, D = tok_ref.shape
    oh3 = (xb[..., None]
           == jax.lax.broadcasted_iota(jnp.int32, (Bb, S, V), 2)).astype(jnp.float32)
    oh = oh3.reshape(Bb * S, V)
    tok = tok_ref[...]
    t = jax.lax.dot_general(oh, tok, (((1,), (0,)), ((), ())),
                            preferred_element_type=jnp.float32)
    out_ref[...] = t.reshape(Bb, S, D) + pos_ref[...][None]


def kernel(x, token_emb, pos_emb):
    x = x.astype(jnp.int32)
    B, S = x.shape
    V, D = token_emb.shape
    grid = (B // _BATCH_BLK,)
    return pl.pallas_call(
        _body,
        grid=grid,
        in_specs=[
            pl.BlockSpec((_BATCH_BLK, S), lambda i: (i, 0)),
            pl.BlockSpec((V, D), lambda i: (0, 0)),
            pl.BlockSpec((S, D), lambda i: (0, 0)),
        ],
        out_specs=pl.BlockSpec((_BATCH_BLK, S, D), lambda i: (i, 0, 0)),
        out_shape=jax.ShapeDtypeStruct((B, S, D), jnp.float32),
    )(x, token_emb, pos_emb)
